# Initial kernel scaffold; baseline (speedup 1.0000x reference)
import functools

import jax
import jax.numpy as jnp
from jax import lax
from jax.experimental import pallas as pl
from jax.experimental.pallas import tpu as pltpu
from jax.experimental.pallas import tpu_sc as plsc

DIM = 64
NC = 2
NS = 16
NW = NC * NS


def _make_gather(batch: int, chunk: int):
    b_per_w = batch // NW
    n_chunks = b_per_w // chunk
    assert b_per_w % chunk == 0

    mesh = plsc.VectorSubcoreMesh(core_axis_name="c", subcore_axis_name="s")

    @functools.partial(
        pl.kernel,
        mesh=mesh,
        out_type=jax.ShapeDtypeStruct((batch, DIM), jnp.float32),
        scratch_types=[
            pltpu.VMEM((chunk,), jnp.int32),
            pltpu.VMEM((chunk, DIM), jnp.float32),
            pltpu.SemaphoreType.DMA,
        ],
    )
    def gather(idx_hbm, table_hbm, out_hbm, idx_v, rows_v, sem):
        wid = lax.axis_index("s") * NC + lax.axis_index("c")
        base = wid * b_per_w

        def body(i, carry):
            off = base + i * chunk
            pltpu.sync_copy(idx_hbm.at[pl.ds(off, chunk)], idx_v)
            pltpu.async_copy(table_hbm.at[idx_v], rows_v, sem).wait()
            pltpu.sync_copy(rows_v, out_hbm.at[pl.ds(off, chunk)])
            return carry

        lax.fori_loop(0, n_chunks, body, 0)

    return gather


def kernel(tokens, table):
    b, h = tokens.shape
    idx = tokens.reshape(-1).astype(jnp.int32)
    out = _make_gather(b * h, 512)(idx, table)
    return out.reshape(b, h, DIM)


# speedup vs baseline: 1.7963x; 1.7963x over previous
import functools

import jax
import jax.numpy as jnp
from jax import lax
from jax.experimental import pallas as pl
from jax.experimental.pallas import tpu as pltpu
from jax.experimental.pallas import tpu_sc as plsc

DIM = 64
NC = 2
NS = 16
NW = NC * NS


def _make_gather(batch: int, chunk: int):
    b_per_w = batch // NW
    n_chunks = b_per_w // chunk
    assert b_per_w % chunk == 0

    mesh = plsc.VectorSubcoreMesh(core_axis_name="c", subcore_axis_name="s")

    @functools.partial(
        pl.kernel,
        mesh=mesh,
        out_type=jax.ShapeDtypeStruct((batch, DIM), jnp.float32),
        compiler_params=pltpu.CompilerParams(use_tc_tiling_on_sc=False),
        scratch_types=[
            pltpu.VMEM((chunk,), jnp.int32),
            pltpu.VMEM((chunk, DIM), jnp.float32),
            pltpu.SemaphoreType.DMA,
        ],
    )
    def gather(idx_hbm, table_hbm, out_hbm, idx_v, rows_v, sem):
        wid = lax.axis_index("s") * NC + lax.axis_index("c")
        base = wid * b_per_w

        def body(i, carry):
            off = base + i * chunk
            pltpu.sync_copy(idx_hbm.at[pl.ds(off, chunk)], idx_v)
            pltpu.async_copy(table_hbm.at[idx_v], rows_v, sem).wait()
            pltpu.sync_copy(rows_v, out_hbm.at[pl.ds(off, chunk)])
            return carry

        lax.fori_loop(0, n_chunks, body, 0)

    return gather


def kernel(tokens, table):
    b, h = tokens.shape
    idx = tokens.reshape(-1).astype(jnp.int32)
    out = _make_gather(b * h, 512)(idx, table)
    return out.reshape(b, h, DIM)


# speedup vs baseline: 1.8722x; 1.0423x over previous
import functools

import jax
import jax.numpy as jnp
from jax import lax
from jax.experimental import pallas as pl
from jax.experimental.pallas import tpu as pltpu
from jax.experimental.pallas import tpu_sc as plsc

DIM = 64
NC = 2
NS = 16
NW = NC * NS


def _make_gather(batch: int, chunk: int):
    b_per_w = batch // NW
    n_chunks = b_per_w // chunk
    assert b_per_w % chunk == 0 and n_chunks % 2 == 0

    mesh = plsc.VectorSubcoreMesh(core_axis_name="c", subcore_axis_name="s")

    @functools.partial(
        pl.kernel,
        mesh=mesh,
        out_type=jax.ShapeDtypeStruct((batch, DIM), jnp.float32),
        compiler_params=pltpu.CompilerParams(use_tc_tiling_on_sc=False),
        scratch_types=[
            pltpu.VMEM((b_per_w,), jnp.int32),
            pltpu.VMEM((chunk, DIM), jnp.float32),
            pltpu.VMEM((chunk, DIM), jnp.float32),
            pltpu.SemaphoreType.DMA,
            pltpu.SemaphoreType.DMA,
            pltpu.SemaphoreType.DMA,
            pltpu.SemaphoreType.DMA,
        ],
    )
    def gather(idx_hbm, table_hbm, out_hbm, idx_v, rows0, rows1,
               sg0, sg1, sw0, sw1):
        wid = lax.axis_index("s") * NC + lax.axis_index("c")
        base = wid * b_per_w
        pltpu.sync_copy(idx_hbm.at[pl.ds(base, b_per_w)], idx_v)

        rows = (rows0, rows1)
        sg = (sg0, sg1)
        sw = (sw0, sw1)

        def do_chunk(i, b, first):
            @pl.when(jnp.logical_not(first))
            def _():
                pltpu.make_async_copy(rows[b], out_hbm.at[pl.ds(base, chunk)],
                                      sw[b]).wait()
            idx_sl = idx_v.at[pl.ds(i * chunk, chunk)]
            pltpu.async_copy(table_hbm.at[idx_sl], rows[b], sg[b]).wait()
            pltpu.async_copy(rows[b], out_hbm.at[pl.ds(base + i * chunk, chunk)],
                             sw[b])

        def body(g, carry):
            first = g == 0
            do_chunk(2 * g, 0, first)
            do_chunk(2 * g + 1, 1, first)
            return carry

        lax.fori_loop(0, n_chunks // 2, body, 0)
        pltpu.make_async_copy(rows0, out_hbm.at[pl.ds(base, chunk)], sw0).wait()
        pltpu.make_async_copy(rows1, out_hbm.at[pl.ds(base, chunk)], sw1).wait()

    return gather


def kernel(tokens, table):
    b, h = tokens.shape
    idx = tokens.reshape(-1).astype(jnp.int32)
    out = _make_gather(b * h, 800)(idx, table)
    return out.reshape(b, h, DIM)
